# baseline (speedup 1.0000x reference)
import functools

import jax
import jax.numpy as jnp
from jax.experimental import pallas as pl
from jax.experimental.pallas import tpu as pltpu

B, T, D = 4, 2048, 4096
E, R = 8, 64
OUT3 = 4096 * 3
ALPHA = 128.0


POOL_TT = 1024
N_POOL_T = T // POOL_TT


def _pool_route_body(tok_ref, wg_ref, logits_ref, sel_ref, ew_ref, imp_ref,
                     load_ref, w_ref):
    b = pl.program_id(0)
    t = pl.program_id(1)

    @pl.when(t == 0)
    def _init():
        logits_ref[b, :] = jnp.zeros((E,), jnp.float32)

    colsum = jnp.sum(tok_ref[0], axis=0, keepdims=True)
    partial = jnp.dot(colsum, wg_ref[...],
                      preferred_element_type=jnp.float32)
    logits_ref[b, :] += partial[0] * (1.0 / T)

    @pl.when((b == B - 1) & (t == N_POOL_T - 1))
    def _route():
        logits = logits_ref[...]
        m = jnp.max(logits, axis=1, keepdims=True)
        ex = jnp.exp(logits - m)
        probs = ex / jnp.sum(ex, axis=1, keepdims=True)
        top1 = jnp.max(probs, axis=1, keepdims=True)
        eidx = jax.lax.broadcasted_iota(jnp.int32, (B, E), 1)
        is_max = probs >= top1
        sel = jnp.min(jnp.where(is_max, eidx, E), axis=1, keepdims=True)
        onehot = (eidx == sel).astype(jnp.float32)
        ew = onehot * top1
        sel_ref[...] = sel
        ew_ref[...] = ew
        imp_ref[...] = jnp.sum(ew, axis=0, keepdims=True)
        load_ref[...] = jnp.sum(onehot, axis=0, keepdims=True) * (1.0 / B)
        w_ref[...] = top1 * (ALPHA / R)


def _pool_route(tokens, Wg):
    fixed = lambda b, t: (0, 0)
    return pl.pallas_call(
        _pool_route_body,
        grid=(B, N_POOL_T),
        in_specs=[
            pl.BlockSpec((1, POOL_TT, D), lambda b, t: (b, t, 0)),
            pl.BlockSpec((D, E), fixed),
        ],
        out_specs=[
            pl.BlockSpec((B, E), fixed),
            pl.BlockSpec((B, 1), fixed),
            pl.BlockSpec((B, E), fixed),
            pl.BlockSpec((1, E), fixed),
            pl.BlockSpec((1, E), fixed),
            pl.BlockSpec((B, 1), fixed),
        ],
        out_shape=(
            jax.ShapeDtypeStruct((B, E), jnp.float32),
            jax.ShapeDtypeStruct((B, 1), jnp.int32),
            jax.ShapeDtypeStruct((B, E), jnp.float32),
            jax.ShapeDtypeStruct((1, E), jnp.float32),
            jax.ShapeDtypeStruct((1, E), jnp.float32),
            jax.ShapeDtypeStruct((B, 1), jnp.float32),
        ),
    )(tokens, Wg)


UTT = 256
N_UT = T // UTT


def _lora_body(sel_ref, tok_ref, a_ref, bw_ref, w_ref, out_ref):
    b = pl.program_id(0)
    x = tok_ref[0].astype(jnp.bfloat16)
    a = a_ref[0].astype(jnp.bfloat16)
    h = jax.lax.dot_general(x, a, (((1,), (1,)), ((), ())),
                            preferred_element_type=jnp.float32)
    hb = (h * w_ref[b, 0]).astype(jnp.bfloat16)
    bw = bw_ref[0].astype(jnp.bfloat16)
    out_ref[0] = jax.lax.dot_general(hb, bw, (((1,), (1,)), ((), ())),
                                     preferred_element_type=jnp.float32)


def _lora_update(tokens, A, Bw, sel, w):
    grid_spec = pltpu.PrefetchScalarGridSpec(
        num_scalar_prefetch=1,
        grid=(B, N_UT),
        in_specs=[
            pl.BlockSpec((1, UTT, D), lambda b, t, sel_ref: (b, t, 0)),
            pl.BlockSpec((1, R, D), lambda b, t, sel_ref: (sel_ref[b], 0, 0)),
            pl.BlockSpec((1, OUT3, R), lambda b, t, sel_ref: (sel_ref[b], 0, 0)),
            pl.BlockSpec((B, 1), lambda b, t, sel_ref: (0, 0)),
        ],
        out_specs=pl.BlockSpec((1, UTT, OUT3), lambda b, t, sel_ref: (b, t, 0)),
    )
    return pl.pallas_call(
        _lora_body,
        grid_spec=grid_spec,
        out_shape=jax.ShapeDtypeStruct((B, T, OUT3), jnp.float32),
        compiler_params=pltpu.CompilerParams(
            dimension_semantics=("parallel", "parallel"),
        ),
    )(sel, tokens, A, Bw, w)


def kernel(tokens, Wg, A, Bw):
    (router_logits, sel2d, expert_weights,
     imp2d, load2d, w) = _pool_route(tokens, Wg)
    sel = sel2d[:, 0]
    weighted_update = _lora_update(tokens, A, Bw, sel, w)
    return (weighted_update, router_logits, sel2d, expert_weights,
            imp2d[0], load2d[0])


# speedup vs baseline: 1.0106x; 1.0106x over previous
import functools

import jax
import jax.numpy as jnp
from jax.experimental import pallas as pl
from jax.experimental.pallas import tpu as pltpu

B, T, D = 4, 2048, 4096
E, R = 8, 64
OUT3 = 4096 * 3
ALPHA = 128.0


POOL_TT = 1024
N_POOL_T = T // POOL_TT


def _pool_route_body(tok_ref, wg_ref, logits_ref, sel_ref, ew_ref, imp_ref,
                     load_ref, w_ref):
    b = pl.program_id(0)
    t = pl.program_id(1)

    @pl.when(t == 0)
    def _init():
        logits_ref[b, :] = jnp.zeros((E,), jnp.float32)

    colsum = jnp.sum(tok_ref[0], axis=0, keepdims=True)
    partial = jnp.dot(colsum, wg_ref[...],
                      preferred_element_type=jnp.float32)
    logits_ref[b, :] += partial[0] * (1.0 / T)

    @pl.when((b == B - 1) & (t == N_POOL_T - 1))
    def _route():
        logits = logits_ref[...]
        m = jnp.max(logits, axis=1, keepdims=True)
        ex = jnp.exp(logits - m)
        probs = ex / jnp.sum(ex, axis=1, keepdims=True)
        top1 = jnp.max(probs, axis=1, keepdims=True)
        eidx = jax.lax.broadcasted_iota(jnp.int32, (B, E), 1)
        is_max = probs >= top1
        sel = jnp.min(jnp.where(is_max, eidx, E), axis=1, keepdims=True)
        onehot = (eidx == sel).astype(jnp.float32)
        ew = onehot * top1
        sel_ref[...] = sel
        ew_ref[...] = ew
        imp_ref[...] = jnp.sum(ew, axis=0, keepdims=True)
        load_ref[...] = jnp.sum(onehot, axis=0, keepdims=True) * (1.0 / B)
        w_ref[...] = top1 * (ALPHA / R)


def _pool_route(tokens, Wg):
    fixed = lambda b, t: (0, 0)
    return pl.pallas_call(
        _pool_route_body,
        grid=(B, N_POOL_T),
        in_specs=[
            pl.BlockSpec((1, POOL_TT, D), lambda b, t: (b, t, 0)),
            pl.BlockSpec((D, E), fixed),
        ],
        out_specs=[
            pl.BlockSpec((B, E), fixed),
            pl.BlockSpec((B, 1), fixed),
            pl.BlockSpec((B, E), fixed),
            pl.BlockSpec((1, E), fixed),
            pl.BlockSpec((1, E), fixed),
            pl.BlockSpec((B, 1), fixed),
        ],
        out_shape=(
            jax.ShapeDtypeStruct((B, E), jnp.float32),
            jax.ShapeDtypeStruct((B, 1), jnp.int32),
            jax.ShapeDtypeStruct((B, E), jnp.float32),
            jax.ShapeDtypeStruct((1, E), jnp.float32),
            jax.ShapeDtypeStruct((1, E), jnp.float32),
            jax.ShapeDtypeStruct((B, 1), jnp.float32),
        ),
    )(tokens, Wg)


UTT = 256
N_UT = T // UTT
NSTEP = B * N_UT
NBUF = 3


def _copy(obuf, slot, out_hbm, bj, tj, sem):
    return pltpu.make_async_copy(
        obuf.at[slot],
        out_hbm.at[bj, pl.ds(tj * UTT, UTT), :],
        sem.at[slot])


def _lora_body(sel_ref, tok_ref, a_ref, bw_ref, w_ref, out_hbm, obuf, sem):
    b = pl.program_id(0)
    t = pl.program_id(1)
    i = b * N_UT + t
    slot = jax.lax.rem(i, NBUF)

    @pl.when(i >= NBUF)
    def _reclaim():
        j = i - NBUF
        _copy(obuf, slot, out_hbm, j // N_UT, jax.lax.rem(j, N_UT), sem).wait()

    x = tok_ref[0].astype(jnp.bfloat16)
    a = a_ref[0].astype(jnp.bfloat16)
    h = jax.lax.dot_general(x, a, (((1,), (1,)), ((), ())),
                            preferred_element_type=jnp.float32)
    hb = (h * w_ref[b, 0]).astype(jnp.bfloat16)
    bw = bw_ref[0].astype(jnp.bfloat16)
    obuf[slot] = jax.lax.dot_general(hb, bw, (((1,), (1,)), ((), ())),
                                    preferred_element_type=jnp.float32)
    _copy(obuf, slot, out_hbm, b, t, sem).start()

    @pl.when(i == NSTEP - 1)
    def _drain():
        for j in range(NSTEP - NBUF, NSTEP):
            bj, tj = divmod(j, N_UT)
            _copy(obuf, j % NBUF, out_hbm, bj, tj, sem).wait()


def _lora_update(tokens, A, Bw, sel, w):
    grid_spec = pltpu.PrefetchScalarGridSpec(
        num_scalar_prefetch=1,
        grid=(B, N_UT),
        in_specs=[
            pl.BlockSpec((1, UTT, D), lambda b, t, sel_ref: (b, t, 0)),
            pl.BlockSpec((1, R, D), lambda b, t, sel_ref: (sel_ref[b], 0, 0)),
            pl.BlockSpec((1, OUT3, R), lambda b, t, sel_ref: (sel_ref[b], 0, 0)),
            pl.BlockSpec((B, 1), lambda b, t, sel_ref: (0, 0)),
        ],
        out_specs=pl.BlockSpec(memory_space=pltpu.MemorySpace.HBM),
        scratch_shapes=[
            pltpu.VMEM((NBUF, UTT, OUT3), jnp.float32),
            pltpu.SemaphoreType.DMA((NBUF,)),
        ],
    )
    return pl.pallas_call(
        _lora_body,
        grid_spec=grid_spec,
        out_shape=jax.ShapeDtypeStruct((B, T, OUT3), jnp.float32),
        compiler_params=pltpu.CompilerParams(
            dimension_semantics=("arbitrary", "arbitrary"),
        ),
    )(sel, tokens, A, Bw, w)


def kernel(tokens, Wg, A, Bw):
    (router_logits, sel2d, expert_weights,
     imp2d, load2d, w) = _pool_route(tokens, Wg)
    sel = sel2d[:, 0]
    weighted_update = _lora_update(tokens, A, Bw, sel, w)
    return (weighted_update, router_logits, sel2d, expert_weights,
            imp2d[0], load2d[0])
